# baseline (speedup 1.0000x reference)
import functools

import jax
import jax.numpy as jnp
from jax.experimental import pallas as pl
from jax.experimental.pallas import tpu as pltpu


def _packed_mlp_kernel(xp_ref, w1_ref, w2_ref, w3_ref, b1_ref, b2_ref,
                       b3_ref, o_ref, *, n_groups):
    xp = xp_ref[...]
    w2 = w2_ref[...]
    b1 = b1_ref[...]
    b2 = b2_ref[...]
    gw = w1_ref.shape[1] // n_groups
    acc = None
    for g in range(n_groups):
        h = jnp.dot(xp, w1_ref[:, g * gw:(g + 1) * gw],
                    preferred_element_type=jnp.float32)
        h = jnp.maximum(h + b1, 0.0)
        h = jnp.dot(h, w2, preferred_element_type=jnp.float32)
        h = jnp.maximum(h + b2, 0.0)
        t = jnp.dot(h, w3_ref[g * gw:(g + 1) * gw, :],
                    preferred_element_type=jnp.float32)
        acc = t if acc is None else acc + t
    z = acc + b3_ref[...]
    o_ref[...] = 1.0 / (1.0 + jnp.exp(-z))


def _simple_mlp_kernel(x_ref, w1_ref, b1_ref, w2_ref, b2_ref, w3_ref,
                       b3_ref, o_ref):
    x = x_ref[...]
    h = jnp.dot(x, w1_ref[...], preferred_element_type=jnp.float32)
    h = jnp.maximum(h + b1_ref[...], 0.0)
    h = jnp.dot(h, w2_ref[...], preferred_element_type=jnp.float32)
    h = jnp.maximum(h + b2_ref[...], 0.0)
    logits = (jnp.dot(h, w3_ref[...], preferred_element_type=jnp.float32)
              + b3_ref[...])
    m = jnp.max(logits, axis=1, keepdims=True)
    e = jnp.exp(logits - m)
    o_ref[...] = (e / jnp.sum(e, axis=1, keepdims=True)).astype(o_ref.dtype)


def _round_up(n, m):
    return ((n + m - 1) // m) * m


def _fallback_forward(x, w1, b1, w2, b2, w3, b3):
    B, F = x.shape
    out_dim = w3.shape[1]
    tb = 2048
    padded_b = _round_up(B, tb)
    if padded_b != B:
        x = jnp.pad(x, ((0, padded_b - B), (0, 0)))
    grid = (padded_b // tb,)

    def rep(arr):
        nd = arr.ndim
        return pl.BlockSpec(arr.shape, lambda i, _n=nd: (0,) * _n)

    out = pl.pallas_call(
        _simple_mlp_kernel,
        out_shape=jax.ShapeDtypeStruct((padded_b, out_dim), jnp.float32),
        grid_spec=pl.GridSpec(
            grid=grid,
            in_specs=[
                pl.BlockSpec((tb, F), lambda i: (i, 0)),
                rep(w1), rep(b1), rep(w2), rep(b2), rep(w3), rep(b3),
            ],
            out_specs=pl.BlockSpec((tb, out_dim), lambda i: (i, 0)),
        ),
        compiler_params=pltpu.CompilerParams(
            dimension_semantics=("parallel",),
        ),
    )(x, w1, b1, w2, b2, w3, b3)
    return out[:B]


def kernel(x, w1, b1, w2, b2, w3, b3):
    B, F = x.shape
    h1 = w1.shape[1]
    h2 = w2.shape[1]
    out_dim = w3.shape[1]

    packable = (
        out_dim == 2 and F > 0 and 128 % F == 0 and h1 == h2
        and h1 > 0 and 128 % h1 == 0 and (128 // F) * h1 % 128 == 0
    )
    if not packable:
        return _fallback_forward(x, w1, b1, w2, b2, w3, b3)

    G = 128 // F
    bpg = 128 // h1
    n_groups = G // bpg

    R = _round_up(B, G) // G
    rows_blk = 2048 if R % 2048 == 0 else (R if R <= 2048 else None)
    if rows_blk is None:
        for cand in (1024, 512, 256, 128, 64, 32, 16, 8):
            if R % cand == 0:
                rows_blk = cand
                break
        else:
            rows_blk = 2048
    R_pad = _round_up(R, rows_blk)
    if R_pad * G != B:
        x = jnp.pad(x, ((0, R_pad * G - B), (0, 0)))
    xp = x.reshape(R_pad, G * F)

    eyeG = jnp.eye(G, dtype=jnp.float32)
    w1_all = jnp.einsum('bB,fh->bfBh', eyeG, w1).reshape(G * F, G * h1)
    eyeB = jnp.eye(bpg, dtype=jnp.float32)
    w2d = jnp.einsum('ij,hk->ihjk', eyeB, w2).reshape(bpg * h1, bpg * h2)
    w3diff = jnp.stack([w3[:, 0] - w3[:, 1], w3[:, 1] - w3[:, 0]], axis=1)
    w3_all = jnp.einsum('bB,hc->bhBc', eyeG, w3diff).reshape(G * h2, G * 2)
    b1t = jnp.tile(b1, (1, bpg))
    b2t = jnp.tile(b2, (1, bpg))
    b3diff = jnp.stack([b3[0, 0] - b3[0, 1], b3[0, 1] - b3[0, 0]])
    b3t = jnp.tile(b3diff.reshape(1, 2), (1, G))

    grid = (R_pad // rows_blk,)

    def rep(arr):
        nd = arr.ndim
        return pl.BlockSpec(arr.shape, lambda i, _n=nd: (0,) * _n)

    flops = 2 * R_pad * (128 * G * h1 + n_groups * 128 * 128 + G * h2 * G * 2)
    bytes_accessed = 4 * (R_pad * 128 + R_pad * G * 2
                          + w1_all.size + w2d.size + w3_all.size + 3 * 128)

    out_p = pl.pallas_call(
        functools.partial(_packed_mlp_kernel, n_groups=n_groups),
        out_shape=jax.ShapeDtypeStruct((R_pad, G * 2), jnp.float32),
        grid_spec=pl.GridSpec(
            grid=grid,
            in_specs=[
                pl.BlockSpec((rows_blk, G * F), lambda i: (i, 0)),
                rep(w1_all), rep(w2d), rep(w3_all),
                rep(b1t), rep(b2t), rep(b3t),
            ],
            out_specs=pl.BlockSpec((rows_blk, G * 2), lambda i: (i, 0)),
        ),
        compiler_params=pltpu.CompilerParams(
            dimension_semantics=("parallel",),
        ),
        cost_estimate=pl.CostEstimate(
            flops=flops,
            bytes_accessed=bytes_accessed,
            transcendentals=R_pad * G * 2,
        ),
    )(xp, w1_all, w2d, w3_all, b1t, b2t, b3t)

    out = out_p.reshape(R_pad * G, 2)
    return out[:B]


# speedup vs baseline: 2.5209x; 2.5209x over previous
import jax
import jax.numpy as jnp
from jax.experimental import pallas as pl
from jax.experimental.pallas import tpu as pltpu


def _mlp_sig_kernel(x_ref, w1_ref, b1_ref, w2_ref, b2_ref, w3_ref, b3_ref,
                    o_ref):
    x = x_ref[...]
    h = jnp.dot(x, w1_ref[...], preferred_element_type=jnp.float32)
    h = jnp.maximum(h + b1_ref[...], 0.0)
    h = jnp.dot(h, w2_ref[...], preferred_element_type=jnp.float32)
    h = jnp.maximum(h + b2_ref[...], 0.0)
    z = (jnp.dot(h, w3_ref[...], preferred_element_type=jnp.float32)
         + b3_ref[...])
    o_ref[...] = 1.0 / (1.0 + jnp.exp(-z))


def _mlp_softmax_kernel(x_ref, w1_ref, b1_ref, w2_ref, b2_ref, w3_ref, b3_ref,
                        o_ref):
    x = x_ref[...]
    h = jnp.dot(x, w1_ref[...], preferred_element_type=jnp.float32)
    h = jnp.maximum(h + b1_ref[...], 0.0)
    h = jnp.dot(h, w2_ref[...], preferred_element_type=jnp.float32)
    h = jnp.maximum(h + b2_ref[...], 0.0)
    logits = (jnp.dot(h, w3_ref[...], preferred_element_type=jnp.float32)
              + b3_ref[...])
    m = jnp.max(logits, axis=1, keepdims=True)
    e = jnp.exp(logits - m)
    o_ref[...] = (e / jnp.sum(e, axis=1, keepdims=True)).astype(o_ref.dtype)


def _round_up(n, m):
    return ((n + m - 1) // m) * m


def kernel(x, w1, b1, w2, b2, w3, b3):
    B, F = x.shape
    h1 = w1.shape[1]
    h2 = w2.shape[1]
    out_dim = w3.shape[1]

    two_class = out_dim == 2
    if two_class:
        w3k = jnp.stack([w3[:, 0] - w3[:, 1], w3[:, 1] - w3[:, 0]], axis=1)
        b3k = jnp.stack([b3[0, 0] - b3[0, 1], b3[0, 1] - b3[0, 0]]).reshape(1, 2)
        body = _mlp_sig_kernel
    else:
        w3k, b3k = w3, b3
        body = _mlp_softmax_kernel

    tb = 8192
    padded_b = _round_up(B, tb)
    if padded_b != B:
        x = jnp.pad(x, ((0, padded_b - B), (0, 0)))
    grid = (padded_b // tb,)

    def rep(arr):
        nd = arr.ndim
        return pl.BlockSpec(arr.shape, lambda i, _n=nd: (0,) * _n)

    flops = 2 * padded_b * (F * h1 + h1 * h2 + h2 * out_dim)
    bytes_accessed = 4 * (padded_b * (F + out_dim)
                          + w1.size + b1.size + w2.size + b2.size
                          + w3k.size + b3k.size)

    out = pl.pallas_call(
        body,
        out_shape=jax.ShapeDtypeStruct((padded_b, out_dim), jnp.float32),
        grid_spec=pl.GridSpec(
            grid=grid,
            in_specs=[
                pl.BlockSpec((tb, F), lambda i: (i, 0)),
                rep(w1), rep(b1), rep(w2), rep(b2), rep(w3k), rep(b3k),
            ],
            out_specs=pl.BlockSpec((tb, out_dim), lambda i: (i, 0)),
        ),
        compiler_params=pltpu.CompilerParams(
            dimension_semantics=("parallel",),
        ),
        cost_estimate=pl.CostEstimate(
            flops=flops,
            bytes_accessed=bytes_accessed,
            transcendentals=padded_b * out_dim,
        ),
    )(x, w1, b1, w2, b2, w3k, b3k)

    if padded_b != B:
        out = out[:B]
    return out


# speedup vs baseline: 26.1193x; 10.3609x over previous
import jax
import jax.numpy as jnp
from jax.experimental import pallas as pl
from jax.experimental.pallas import tpu as pltpu

_CHUNKS = 4


def _packed_kernel(x0_ref, x1_ref, x2_ref, x3_ref, w1_ref, b1_ref, w2_ref,
                   b2_ref, w3_ref, b3_ref, o_ref):
    x16 = jnp.concatenate(
        [x0_ref[...], x1_ref[...], x2_ref[...], x3_ref[...]], axis=0)
    h = jnp.dot(w1_ref[...], x16, preferred_element_type=jnp.float32)
    h = jnp.maximum(h + b1_ref[...], 0.0)
    h = jnp.dot(w2_ref[...], h, preferred_element_type=jnp.float32)
    h = jnp.maximum(h + b2_ref[...], 0.0)
    z = (jnp.dot(w3_ref[...], h, preferred_element_type=jnp.float32)
         + b3_ref[...])
    o_ref[...] = 1.0 / (1.0 + jnp.exp(-z))


def _softmax_kernel(xt_ref, w1_ref, b1_ref, w2_ref, b2_ref, w3_ref, b3_ref,
                    o_ref):
    h = jnp.dot(w1_ref[...], xt_ref[...], preferred_element_type=jnp.float32)
    h = jnp.maximum(h + b1_ref[...], 0.0)
    h = jnp.dot(w2_ref[...], h, preferred_element_type=jnp.float32)
    h = jnp.maximum(h + b2_ref[...], 0.0)
    logits = (jnp.dot(w3_ref[...], h, preferred_element_type=jnp.float32)
              + b3_ref[...])
    m = jnp.max(logits, axis=0, keepdims=True)
    e = jnp.exp(logits - m)
    o_ref[...] = (e / jnp.sum(e, axis=0, keepdims=True)).astype(o_ref.dtype)


def _round_up(n, m):
    return ((n + m - 1) // m) * m


def _blockdiag(m, copies):
    eye = jnp.eye(copies, dtype=m.dtype)
    r, c = m.shape
    return jnp.einsum('ij,rc->irjc', eye, m).reshape(copies * r, copies * c)


def _general_forward(x, w1, b1, w2, b2, w3, b3):
    B, F = x.shape
    h1, h2, out_dim = w1.shape[1], w2.shape[1], w3.shape[1]
    tb = 4096
    padded_b = _round_up(B, tb)
    xt = x.T
    if padded_b != B:
        xt = jnp.pad(xt, ((0, 0), (0, padded_b - B)))
    w1t, w2t, w3t = w1.T, w2.T, w3.T
    b1t = b1.reshape(h1, 1)
    b2t = b2.reshape(h2, 1)
    b3t = b3.reshape(out_dim, 1)

    def rep(arr):
        nd = arr.ndim
        return pl.BlockSpec(arr.shape, lambda i, _n=nd: (0,) * _n)

    out_t = pl.pallas_call(
        _softmax_kernel,
        out_shape=jax.ShapeDtypeStruct((out_dim, padded_b), jnp.float32),
        grid_spec=pl.GridSpec(
            grid=(padded_b // tb,),
            in_specs=[
                pl.BlockSpec((F, tb), lambda i: (0, i)),
                rep(w1t), rep(b1t), rep(w2t), rep(b2t), rep(w3t), rep(b3t),
            ],
            out_specs=pl.BlockSpec((out_dim, tb), lambda i: (0, i)),
        ),
        compiler_params=pltpu.CompilerParams(
            dimension_semantics=("parallel",),
        ),
    )(xt, w1t, b1t, w2t, b2t, w3t, b3t)
    return out_t[:, :B].T


def kernel(x, w1, b1, w2, b2, w3, b3):
    B, F = x.shape
    h1 = w1.shape[1]
    h2 = w2.shape[1]
    out_dim = w3.shape[1]

    if out_dim != 2 or F != 4 or h1 != 32 or h2 != 32:
        return _general_forward(x, w1, b1, w2, b2, w3, b3)

    tb = 4096
    padded_b = _round_up(B, _CHUNKS * tb) if B % (_CHUNKS * tb) else B
    xt = x.T
    if padded_b != B:
        xt = jnp.pad(xt, ((0, 0), (0, padded_b - B)))
    bq = padded_b // _CHUNKS
    steps = bq // tb

    w1b = _blockdiag(w1.T, _CHUNKS)
    w2b = _blockdiag(w2.T, _CHUNKS)
    w3d = jnp.stack([w3[:, 0] - w3[:, 1], w3[:, 1] - w3[:, 0]], axis=1)
    w3b = _blockdiag(w3d.T, _CHUNKS)
    b1b = jnp.tile(b1.reshape(h1, 1), (_CHUNKS, 1))
    b2b = jnp.tile(b2.reshape(h2, 1), (_CHUNKS, 1))
    b3d = jnp.stack([b3[0, 0] - b3[0, 1], b3[0, 1] - b3[0, 0]])
    b3b = jnp.tile(b3d.reshape(2, 1), (_CHUNKS, 1))

    def rep(arr):
        nd = arr.ndim
        return pl.BlockSpec(arr.shape, lambda i, _n=nd: (0,) * _n)

    def chunk_spec(c):
        return pl.BlockSpec((F, tb), lambda i, _c=c: (0, _c * steps + i))

    flops = 2 * padded_b * (F * h1 + h1 * h2 + h2 * 2)
    bytes_accessed = 4 * (padded_b * (F + 2) + w1b.size + w2b.size
                          + w3b.size + 3 * 128)

    out_p = pl.pallas_call(
        _packed_kernel,
        out_shape=jax.ShapeDtypeStruct((2 * _CHUNKS, bq), jnp.float32),
        grid_spec=pl.GridSpec(
            grid=(steps,),
            in_specs=[
                chunk_spec(0), chunk_spec(1), chunk_spec(2), chunk_spec(3),
                rep(w1b), rep(b1b), rep(w2b), rep(b2b), rep(w3b), rep(b3b),
            ],
            out_specs=pl.BlockSpec((2 * _CHUNKS, tb), lambda i: (0, i)),
        ),
        compiler_params=pltpu.CompilerParams(
            dimension_semantics=("parallel",),
        ),
        cost_estimate=pl.CostEstimate(
            flops=flops,
            bytes_accessed=bytes_accessed,
            transcendentals=padded_b * 2,
        ),
    )(xt, xt, xt, xt, w1b, b1b, w2b, b2b, w3b, b3b)

    out = out_p.reshape(_CHUNKS, 2, bq).transpose(0, 2, 1).reshape(padded_b, 2)
    if padded_b != B:
        out = out[:B]
    return out
